# baseline (speedup 1.0000x reference)
import functools

import jax
import jax.numpy as jnp
from jax import lax
from jax.experimental import pallas as pl
from jax.experimental.pallas import tpu as pltpu
from jax.experimental.pallas import tpu_sc as plsc

N = 10000
K = 16
D = 128
NUM_COEF = 9
ROW = NUM_COEF * D

NW = 32
NPAD = 10240
PER_W = NPAD // NW
CHUNK = 2
NCHUNK = PER_W // CHUNK
NE = N * K


def _stage_a_body(x_ref, al_ref, w1_ref, b1_ref, g_ref, bln_ref, w2_ref,
                  b2_ref, o_ref):
    x = x_ref[...]
    h = jnp.dot(x, w1_ref[...], preferred_element_type=jnp.float32)
    h = h + b1_ref[...]
    mu = jnp.mean(h, axis=-1, keepdims=True)
    var = jnp.mean((h - mu) ** 2, axis=-1, keepdims=True)
    h = (h - mu) * lax.rsqrt(var + 1e-5) * g_ref[...] + bln_ref[...]
    h = h * jax.nn.sigmoid(h)
    ih = jnp.dot(h, w2_ref[...], preferred_element_type=jnp.float32)
    ih = ih + b2_ref[...]
    hrow = lax.broadcasted_iota(jnp.int32, (8, D), 0)
    hcol = lax.broadcasted_iota(jnp.int32, (8, D), 1) // 16
    expand = (hrow == hcol).astype(jnp.float32)
    o_ref[...] = jnp.dot(al_ref[...], expand,
                         preferred_element_type=jnp.float32) * ih


def _stage_a(x2, al, w1, b1, g, bln, w2, b2):
    R = 2000
    grid = NE // R
    return pl.pallas_call(
        _stage_a_body,
        grid=(grid,),
        in_specs=[
            pl.BlockSpec((R, 16), lambda i: (i, 0)),
            pl.BlockSpec((R, 8), lambda i: (i, 0)),
            pl.BlockSpec((16, 64), lambda i: (0, 0)),
            pl.BlockSpec((1, 64), lambda i: (0, 0)),
            pl.BlockSpec((1, 64), lambda i: (0, 0)),
            pl.BlockSpec((1, 64), lambda i: (0, 0)),
            pl.BlockSpec((64, D), lambda i: (0, 0)),
            pl.BlockSpec((1, D), lambda i: (0, 0)),
        ],
        out_specs=pl.BlockSpec((R, D), lambda i: (i, 0)),
        out_shape=jax.ShapeDtypeStruct((NE, D), jnp.float32),
    )(x2, al, w1, b1, g, bln, w2, b2)


def _sc_gather_reduce(value2, idxf, a_pad):
    mesh = plsc.VectorSubcoreMesh(core_axis_name="c", subcore_axis_name="s")

    @functools.partial(
        pl.kernel,
        out_type=jax.ShapeDtypeStruct((NPAD, ROW), jnp.float32),
        mesh=mesh,
        scratch_types=[
            pltpu.VMEM((PER_W * K,), jnp.int32),
            pltpu.VMEM((CHUNK * K, D), jnp.float32),
            pltpu.VMEM((CHUNK * K, D), jnp.float32),
            pltpu.VMEM((CHUNK * K, ROW), jnp.float32),
            pltpu.VMEM((CHUNK * K, ROW), jnp.float32),
            pltpu.VMEM((CHUNK, ROW), jnp.float32),
            pltpu.VMEM((CHUNK, ROW), jnp.float32),
            pltpu.SemaphoreType.DMA,
            pltpu.SemaphoreType.DMA,
            pltpu.SemaphoreType.DMA,
            pltpu.SemaphoreType.DMA,
            pltpu.SemaphoreType.DMA,
            pltpu.SemaphoreType.DMA,
        ],
    )
    def body(value_hbm, idx_hbm, a_hbm, out_hbm,
             idx_all, a_v0, a_v1, v_v0, v_v1, out_v0, out_v1,
             gsem0, gsem1, asem0, asem1, osem0, osem1):
        wid = lax.axis_index("s") * 2 + lax.axis_index("c")
        base = wid * PER_W
        a_v = (a_v0, a_v1)
        v_v = (v_v0, v_v1)
        out_v = (out_v0, out_v1)
        gsem = (gsem0, gsem1)
        asem = (asem0, asem1)
        osem = (osem0, osem1)

        pltpu.sync_copy(idx_hbm.at[pl.ds(base * K, PER_W * K)], idx_all)

        def fetch(b, t):
            off = t * (CHUNK * K)
            pltpu.async_copy(
                value_hbm.at[idx_all.at[pl.ds(off, CHUNK * K)]],
                v_v[b], gsem[b])
            abase = jnp.minimum(base + t * CHUNK, N - CHUNK) * K
            pltpu.async_copy(a_hbm.at[pl.ds(abase, CHUNK * K)], a_v[b],
                             asem[b])

        def accum(b):
            pltpu.make_async_copy(
                value_hbm.at[idx_all.at[pl.ds(0, CHUNK * K)]], v_v[b],
                gsem[b]).wait()
            pltpu.make_async_copy(a_hbm.at[pl.ds(0, CHUNK * K)], a_v[b],
                                  asem[b]).wait()
            for i in range(CHUNK):
                for c8 in range(D // 16):
                    def k_body(k, accs, i=i, c8=c8, b=b):
                        r = i * K + k
                        av = a_v[b][r, pl.ds(c8 * 16, 16)]
                        return tuple(
                            accs[m] + v_v[b][r, pl.ds(m * D + c8 * 16, 16)]
                            * av
                            for m in range(NUM_COEF))
                    accs = lax.fori_loop(
                        0, K, k_body,
                        tuple(jnp.zeros((16,), jnp.float32)
                              for _ in range(NUM_COEF)))
                    for m in range(NUM_COEF):
                        out_v[b][i, pl.ds(m * D + c8 * 16, 16)] = accs[m]

        def writeback(b, t):
            pltpu.async_copy(out_v[b],
                             out_hbm.at[pl.ds(base + t * CHUNK, CHUNK)],
                             osem[b])

        def drain_out(b):
            pltpu.make_async_copy(out_v[b], out_hbm.at[pl.ds(0, CHUNK)],
                                  osem[b]).wait()

        fetch(0, 0)
        fetch(1, 1)
        for b in range(2):
            accum(b)
            writeback(b, b)
            fetch(b, b + 2)

        def pair_body(p, carry):
            for b in range(2):
                t = 2 * p + b
                drain_out(b)
                accum(b)
                writeback(b, t)
                fetch(b, t + 2)
            return carry

        lax.fori_loop(1, NCHUNK // 2 - 1, pair_body, 0)

        for b in range(2):
            drain_out(b)
            accum(b)
            writeback(b, NCHUNK - 2 + b)
        for b in range(2):
            drain_out(b)

    return body(value2, idxf, a_pad)


def _stage_c_body(x_ref, w_ref, b_ref, o_ref):
    for m in range(NUM_COEF):
        l = 0 if m == 0 else (1 if m < 4 else 2)
        o = jnp.dot(x_ref[:, m, :], w_ref[l],
                    preferred_element_type=jnp.float32)
        if m == 0:
            o = o + b_ref[...]
        o_ref[:, m, :] = o


def _stage_c(x3, w, b):
    R = 500
    grid = N // R
    return pl.pallas_call(
        _stage_c_body,
        grid=(grid,),
        in_specs=[
            pl.BlockSpec((R, NUM_COEF, D), lambda i: (i, 0, 0)),
            pl.BlockSpec((3, D, D), lambda i: (0, 0, 0)),
            pl.BlockSpec((1, D), lambda i: (0, 0)),
        ],
        out_specs=pl.BlockSpec((R, NUM_COEF, D), lambda i: (i, 0, 0)),
        out_shape=jax.ShapeDtypeStruct((N, NUM_COEF, D), jnp.float32),
    )(x3, w, b)


def kernel(alpha, value, x_edge, node_pos, edge_dis, f_sparse_idx_node,
           rad_w1, rad_b1, rad_ln_g, rad_ln_b, rad_w2, rad_b2, so3_w, so3_b):
    x2 = x_edge.reshape(NE, 16)
    al = alpha.reshape(NE, 8)
    idxf = f_sparse_idx_node.astype(jnp.int32).reshape(NE)
    idxf = jnp.pad(idxf, (0, NPAD * K - NE))
    value2 = value.reshape(N, ROW)

    a_pad = _stage_a(x2, al, rad_w1, rad_b1.reshape(1, 64),
                     rad_ln_g.reshape(1, 64), rad_ln_b.reshape(1, 64),
                     rad_w2, rad_b2.reshape(1, D))
    node_out = _sc_gather_reduce(value2, idxf, a_pad)
    return _stage_c(node_out.reshape(NPAD, NUM_COEF, D), so3_w,
                    so3_b.reshape(1, D))


# speedup vs baseline: 1.1776x; 1.1776x over previous
import functools

import jax
import jax.numpy as jnp
from jax import lax
from jax.experimental import pallas as pl
from jax.experimental.pallas import tpu as pltpu
from jax.experimental.pallas import tpu_sc as plsc

N = 10000
K = 16
D = 128
NUM_COEF = 9
ROW = NUM_COEF * D

NW = 32
NPAD = 10240
PER_W = NPAD // NW
CHUNK = 2
NCHUNK = PER_W // CHUNK
NE = N * K


def _stage_a_body(x_ref, al_ref, w1_ref, b1_ref, g_ref, bln_ref, w2_ref,
                  b2_ref, o_ref):
    x = x_ref[...]
    h = jnp.dot(x, w1_ref[...], preferred_element_type=jnp.float32)
    h = h + b1_ref[...]
    mu = jnp.mean(h, axis=-1, keepdims=True)
    var = jnp.mean((h - mu) ** 2, axis=-1, keepdims=True)
    h = (h - mu) * lax.rsqrt(var + 1e-5) * g_ref[...] + bln_ref[...]
    h = h * jax.nn.sigmoid(h)
    ih = jnp.dot(h, w2_ref[...], preferred_element_type=jnp.float32)
    ih = ih + b2_ref[...]
    hrow = lax.broadcasted_iota(jnp.int32, (8, D), 0)
    hcol = lax.broadcasted_iota(jnp.int32, (8, D), 1) // 16
    expand = (hrow == hcol).astype(jnp.float32)
    o_ref[...] = jnp.dot(al_ref[...], expand,
                         preferred_element_type=jnp.float32) * ih


def _stage_a(x2, al, w1, b1, g, bln, w2, b2):
    R = 2000
    grid = NE // R
    return pl.pallas_call(
        _stage_a_body,
        grid=(grid,),
        in_specs=[
            pl.BlockSpec((R, 16), lambda i: (i, 0)),
            pl.BlockSpec((R, 8), lambda i: (i, 0)),
            pl.BlockSpec((16, 64), lambda i: (0, 0)),
            pl.BlockSpec((1, 64), lambda i: (0, 0)),
            pl.BlockSpec((1, 64), lambda i: (0, 0)),
            pl.BlockSpec((1, 64), lambda i: (0, 0)),
            pl.BlockSpec((64, D), lambda i: (0, 0)),
            pl.BlockSpec((1, D), lambda i: (0, 0)),
        ],
        out_specs=pl.BlockSpec((R, D), lambda i: (i, 0)),
        out_shape=jax.ShapeDtypeStruct((NE, D), jnp.float32),
    )(x2, al, w1, b1, g, bln, w2, b2)


def _sc_gather_reduce(value2, idxf, a_pad):
    mesh = plsc.VectorSubcoreMesh(core_axis_name="c", subcore_axis_name="s")

    @functools.partial(
        pl.kernel,
        out_type=jax.ShapeDtypeStruct((NPAD, ROW), jnp.float32),
        mesh=mesh,
        scratch_types=[
            pltpu.VMEM((CHUNK * K,), jnp.int32),
            pltpu.VMEM((CHUNK * K,), jnp.int32),
            pltpu.VMEM((CHUNK * K, D), jnp.float32),
            pltpu.VMEM((CHUNK * K, D), jnp.float32),
            pltpu.VMEM((CHUNK * K, ROW), jnp.float32),
            pltpu.VMEM((CHUNK * K, ROW), jnp.float32),
            pltpu.VMEM((CHUNK, ROW), jnp.float32),
            pltpu.VMEM((CHUNK, ROW), jnp.float32),
            pltpu.SemaphoreType.DMA,
            pltpu.SemaphoreType.DMA,
            pltpu.SemaphoreType.DMA,
            pltpu.SemaphoreType.DMA,
            pltpu.SemaphoreType.DMA,
            pltpu.SemaphoreType.DMA,
        ],
    )
    def body(value_hbm, idx_hbm, a_hbm, out_hbm,
             idx_v0, idx_v1, a_v0, a_v1, v_v0, v_v1, out_v0, out_v1,
             gsem0, gsem1, asem0, asem1, osem0, osem1):
        wid = lax.axis_index("s") * 2 + lax.axis_index("c")
        base = wid * PER_W
        idx_v = (idx_v0, idx_v1)
        a_v = (a_v0, a_v1)
        v_v = (v_v0, v_v1)
        out_v = (out_v0, out_v1)
        gsem = (gsem0, gsem1)
        asem = (asem0, asem1)
        osem = (osem0, osem1)

        def fetch(b, t):
            rbase = jnp.minimum(base + t * CHUNK, N - CHUNK) * K
            pltpu.sync_copy(idx_hbm.at[pl.ds(rbase, CHUNK * K)], idx_v[b])
            pltpu.async_copy(value_hbm.at[idx_v[b]], v_v[b], gsem[b])
            pltpu.async_copy(a_hbm.at[pl.ds(rbase, CHUNK * K)], a_v[b],
                             asem[b])

        def accum(b):
            pltpu.make_async_copy(value_hbm.at[idx_v[b]], v_v[b],
                                  gsem[b]).wait()
            pltpu.make_async_copy(a_hbm.at[pl.ds(0, CHUNK * K)], a_v[b],
                                  asem[b]).wait()
            for i in range(CHUNK):
                for c8 in range(D // 16):
                    def k_body(k, accs, i=i, c8=c8, b=b):
                        r = i * K + k
                        av = a_v[b][r, pl.ds(c8 * 16, 16)]
                        return tuple(
                            accs[m] + v_v[b][r, pl.ds(m * D + c8 * 16, 16)]
                            * av
                            for m in range(NUM_COEF))
                    accs = lax.fori_loop(
                        0, K, k_body,
                        tuple(jnp.zeros((16,), jnp.float32)
                              for _ in range(NUM_COEF)))
                    for m in range(NUM_COEF):
                        out_v[b][i, pl.ds(m * D + c8 * 16, 16)] = accs[m]

        def writeback(b, t):
            pltpu.async_copy(out_v[b],
                             out_hbm.at[pl.ds(base + t * CHUNK, CHUNK)],
                             osem[b])

        def drain_out(b):
            pltpu.make_async_copy(out_v[b], out_hbm.at[pl.ds(0, CHUNK)],
                                  osem[b]).wait()

        fetch(0, 0)
        fetch(1, 1)
        for b in range(2):
            accum(b)
            writeback(b, b)
            fetch(b, b + 2)

        def pair_body(p, carry):
            for b in range(2):
                t = 2 * p + b
                drain_out(b)
                accum(b)
                writeback(b, t)
                fetch(b, t + 2)
            return carry

        lax.fori_loop(1, NCHUNK // 2 - 1, pair_body, 0)

        for b in range(2):
            drain_out(b)
            accum(b)
            writeback(b, NCHUNK - 2 + b)
        for b in range(2):
            drain_out(b)

    return body(value2, idxf, a_pad)


def _stage_c_body(x_ref, w_ref, b_ref, o_ref):
    for m in range(NUM_COEF):
        l = 0 if m == 0 else (1 if m < 4 else 2)
        o = jnp.dot(x_ref[:, m, :], w_ref[l],
                    preferred_element_type=jnp.float32)
        if m == 0:
            o = o + b_ref[...]
        o_ref[:, m, :] = o


def _stage_c(x3, w, b):
    R = 500
    grid = N // R
    return pl.pallas_call(
        _stage_c_body,
        grid=(grid,),
        in_specs=[
            pl.BlockSpec((R, NUM_COEF, D), lambda i: (i, 0, 0)),
            pl.BlockSpec((3, D, D), lambda i: (0, 0, 0)),
            pl.BlockSpec((1, D), lambda i: (0, 0)),
        ],
        out_specs=pl.BlockSpec((R, NUM_COEF, D), lambda i: (i, 0, 0)),
        out_shape=jax.ShapeDtypeStruct((N, NUM_COEF, D), jnp.float32),
    )(x3, w, b)


def kernel(alpha, value, x_edge, node_pos, edge_dis, f_sparse_idx_node,
           rad_w1, rad_b1, rad_ln_g, rad_ln_b, rad_w2, rad_b2, so3_w, so3_b):
    x2 = x_edge.reshape(NE, 16)
    al = alpha.reshape(NE, 8)
    idxf = f_sparse_idx_node.astype(jnp.int32).reshape(NE)
    value2 = value.reshape(N, ROW)

    a_pad = _stage_a(x2, al, rad_w1, rad_b1.reshape(1, 64),
                     rad_ln_g.reshape(1, 64), rad_ln_b.reshape(1, 64),
                     rad_w2, rad_b2.reshape(1, D))
    node_out = _sc_gather_reduce(value2, idxf, a_pad)
    return _stage_c(node_out.reshape(NPAD, NUM_COEF, D), so3_w,
                    so3_b.reshape(1, D))


# speedup vs baseline: 1.2311x; 1.0455x over previous
import functools

import jax
import jax.numpy as jnp
from jax import lax
from jax.experimental import pallas as pl
from jax.experimental.pallas import tpu as pltpu
from jax.experimental.pallas import tpu_sc as plsc

N = 10000
K = 16
D = 128
NUM_COEF = 9
ROW = NUM_COEF * D

NW = 32
NPAD = 10240
PER_W = NPAD // NW
CHUNK = 2
NCHUNK = PER_W // CHUNK
NE = N * K


def _stage_a_body(x_ref, al_ref, w1_ref, b1_ref, g_ref, bln_ref, w2_ref,
                  b2_ref, o_ref):
    x = x_ref[...]
    h = jnp.dot(x, w1_ref[...], preferred_element_type=jnp.float32)
    h = h + b1_ref[...]
    mu = jnp.mean(h, axis=-1, keepdims=True)
    var = jnp.mean((h - mu) ** 2, axis=-1, keepdims=True)
    h = (h - mu) * lax.rsqrt(var + 1e-5) * g_ref[...] + bln_ref[...]
    h = h * jax.nn.sigmoid(h)
    ih = jnp.dot(h, w2_ref[...], preferred_element_type=jnp.float32)
    ih = ih + b2_ref[...]
    hrow = lax.broadcasted_iota(jnp.int32, (8, D), 0)
    hcol = lax.broadcasted_iota(jnp.int32, (8, D), 1) // 16
    expand = (hrow == hcol).astype(jnp.float32)
    o_ref[...] = jnp.dot(al_ref[...], expand,
                         preferred_element_type=jnp.float32) * ih


def _stage_a(x2, al, w1, b1, g, bln, w2, b2):
    R = 2000
    grid = NE // R
    return pl.pallas_call(
        _stage_a_body,
        grid=(grid,),
        in_specs=[
            pl.BlockSpec((R, 16), lambda i: (i, 0)),
            pl.BlockSpec((R, 8), lambda i: (i, 0)),
            pl.BlockSpec((16, 64), lambda i: (0, 0)),
            pl.BlockSpec((1, 64), lambda i: (0, 0)),
            pl.BlockSpec((1, 64), lambda i: (0, 0)),
            pl.BlockSpec((1, 64), lambda i: (0, 0)),
            pl.BlockSpec((64, D), lambda i: (0, 0)),
            pl.BlockSpec((1, D), lambda i: (0, 0)),
        ],
        out_specs=pl.BlockSpec((R, D), lambda i: (i, 0)),
        out_shape=jax.ShapeDtypeStruct((NE, D), jnp.float32),
    )(x2, al, w1, b1, g, bln, w2, b2)


def _sc_gather_reduce(value2, idxf, a_pad):
    mesh = plsc.VectorSubcoreMesh(core_axis_name="c", subcore_axis_name="s")

    @functools.partial(
        pl.kernel,
        out_type=jax.ShapeDtypeStruct((NPAD, ROW), jnp.float32),
        mesh=mesh,
        scratch_types=[
            pltpu.VMEM((CHUNK * K,), jnp.int32),
            pltpu.VMEM((CHUNK * K,), jnp.int32),
            pltpu.VMEM((CHUNK * K, D), jnp.float32),
            pltpu.VMEM((CHUNK * K, D), jnp.float32),
            pltpu.VMEM((CHUNK * K, ROW), jnp.float32),
            pltpu.VMEM((CHUNK * K, ROW), jnp.float32),
            pltpu.VMEM((CHUNK, ROW), jnp.float32),
            pltpu.VMEM((CHUNK, ROW), jnp.float32),
            pltpu.SemaphoreType.DMA,
            pltpu.SemaphoreType.DMA,
            pltpu.SemaphoreType.DMA,
            pltpu.SemaphoreType.DMA,
            pltpu.SemaphoreType.DMA,
            pltpu.SemaphoreType.DMA,
            pltpu.SemaphoreType.DMA,
            pltpu.SemaphoreType.DMA,
        ],
    )
    def body(value_hbm, idx_hbm, a_hbm, out_hbm,
             idx_v0, idx_v1, a_v0, a_v1, v_v0, v_v1, out_v0, out_v1,
             gsem0, gsem1, asem0, asem1, osem0, osem1, isem0, isem1):
        wid = lax.axis_index("s") * 2 + lax.axis_index("c")
        base = wid * PER_W
        idx_v = (idx_v0, idx_v1)
        a_v = (a_v0, a_v1)
        v_v = (v_v0, v_v1)
        out_v = (out_v0, out_v1)
        gsem = (gsem0, gsem1)
        asem = (asem0, asem1)
        osem = (osem0, osem1)
        isem = (isem0, isem1)

        def clamped(t):
            return jnp.minimum(base + t * CHUNK, N - CHUNK) * K

        def fetch_idx(b, t):
            pltpu.async_copy(idx_hbm.at[pl.ds(clamped(t), CHUNK * K)],
                             idx_v[b], isem[b])

        def fetch(b, t):
            pltpu.make_async_copy(idx_hbm.at[pl.ds(0, CHUNK * K)],
                                  idx_v[b], isem[b]).wait()
            pltpu.async_copy(value_hbm.at[idx_v[b]], v_v[b], gsem[b])
            pltpu.async_copy(a_hbm.at[pl.ds(clamped(t), CHUNK * K)],
                             a_v[b], asem[b])

        def accum(b, t):
            pltpu.make_async_copy(value_hbm.at[idx_v[b]], v_v[b],
                                  gsem[b]).wait()
            fetch_idx(b, t + 2)
            pltpu.make_async_copy(a_hbm.at[pl.ds(0, CHUNK * K)], a_v[b],
                                  asem[b]).wait()
            for i in range(CHUNK):
                for c8 in range(D // 16):
                    def k_body(k, accs, i=i, c8=c8, b=b):
                        r = i * K + k
                        av = a_v[b][r, pl.ds(c8 * 16, 16)]
                        return tuple(
                            accs[m] + v_v[b][r, pl.ds(m * D + c8 * 16, 16)]
                            * av
                            for m in range(NUM_COEF))
                    accs = lax.fori_loop(
                        0, K, k_body,
                        tuple(jnp.zeros((16,), jnp.float32)
                              for _ in range(NUM_COEF)))
                    for m in range(NUM_COEF):
                        out_v[b][i, pl.ds(m * D + c8 * 16, 16)] = accs[m]

        def writeback(b, t):
            pltpu.async_copy(out_v[b],
                             out_hbm.at[pl.ds(base + t * CHUNK, CHUNK)],
                             osem[b])

        def drain_out(b):
            pltpu.make_async_copy(out_v[b], out_hbm.at[pl.ds(0, CHUNK)],
                                  osem[b]).wait()

        fetch_idx(0, 0)
        fetch_idx(1, 1)
        fetch(0, 0)
        fetch(1, 1)
        for b in range(2):
            accum(b, b)
            writeback(b, b)
            fetch(b, b + 2)

        def pair_body(p, carry):
            for b in range(2):
                t = 2 * p + b
                drain_out(b)
                accum(b, t)
                writeback(b, t)
                fetch(b, t + 2)
            return carry

        lax.fori_loop(1, NCHUNK // 2 - 1, pair_body, 0)

        for b in range(2):
            drain_out(b)
            accum(b, NCHUNK - 2 + b)
            writeback(b, NCHUNK - 2 + b)
        for b in range(2):
            pltpu.make_async_copy(idx_hbm.at[pl.ds(0, CHUNK * K)],
                                  idx_v[b], isem[b]).wait()
            drain_out(b)

    return body(value2, idxf, a_pad)


def _stage_c_body(x_ref, w_ref, b_ref, o_ref):
    for m in range(NUM_COEF):
        l = 0 if m == 0 else (1 if m < 4 else 2)
        o = jnp.dot(x_ref[:, m, :], w_ref[l],
                    preferred_element_type=jnp.float32)
        if m == 0:
            o = o + b_ref[...]
        o_ref[:, m, :] = o


def _stage_c(x3, w, b):
    R = 500
    grid = N // R
    return pl.pallas_call(
        _stage_c_body,
        grid=(grid,),
        in_specs=[
            pl.BlockSpec((R, NUM_COEF, D), lambda i: (i, 0, 0)),
            pl.BlockSpec((3, D, D), lambda i: (0, 0, 0)),
            pl.BlockSpec((1, D), lambda i: (0, 0)),
        ],
        out_specs=pl.BlockSpec((R, NUM_COEF, D), lambda i: (i, 0, 0)),
        out_shape=jax.ShapeDtypeStruct((N, NUM_COEF, D), jnp.float32),
    )(x3, w, b)


def kernel(alpha, value, x_edge, node_pos, edge_dis, f_sparse_idx_node,
           rad_w1, rad_b1, rad_ln_g, rad_ln_b, rad_w2, rad_b2, so3_w, so3_b):
    x2 = x_edge.reshape(NE, 16)
    al = alpha.reshape(NE, 8)
    idxf = f_sparse_idx_node.astype(jnp.int32).reshape(NE)
    value2 = value.reshape(N, ROW)

    a_pad = _stage_a(x2, al, rad_w1, rad_b1.reshape(1, 64),
                     rad_ln_g.reshape(1, 64), rad_ln_b.reshape(1, 64),
                     rad_w2, rad_b2.reshape(1, D))
    node_out = _sc_gather_reduce(value2, idxf, a_pad)
    return _stage_c(node_out.reshape(NPAD, NUM_COEF, D), so3_w,
                    so3_b.reshape(1, D))
